# baseline (speedup 1.0000x reference)
import functools

import jax
import jax.numpy as jnp
from jax import lax
from jax.experimental import pallas as pl
from jax.experimental.pallas import tpu as pltpu
from jax.experimental.pallas import tpu_sc as plsc

_LANES = 128
_SUBC = 16
_ROW_BLK = 2000
_NBUF = 4
_CORE0_FRAC = 0.25


def _mesh():
    return plsc.VectorSubcoreMesh(core_axis_name="c", subcore_axis_name="s")


_SC_PARAMS = pltpu.CompilerParams(use_tc_tiling_on_sc=False)


@functools.lru_cache(maxsize=None)
def _make_deg(k0, k1, n_acc):
    zr = n_acc // _SUBC
    kmax = max(k0, k1)

    @functools.partial(
        pl.kernel,
        mesh=_mesh(),
        out_type=[jax.ShapeDtypeStruct((n_acc,), jnp.float32),
                  jax.ShapeDtypeStruct((n_acc,), jnp.float32)],
        compiler_params=_SC_PARAMS,
        scratch_types=[
            pltpu.VMEM((kmax, _LANES), jnp.int32),
            pltpu.VMEM((_LANES,), jnp.float32),
            pltpu.VMEM((zr,), jnp.float32),
            pltpu.VMEM_SHARED((n_acc,), jnp.float32),
        ],
    )
    def deg_kernel(dst0_hbm, dst1_hbm, ones_hbm, zeros_hbm, out0_hbm, out1_hbm,
                   dst_v, ones_v, stage_v, acc_sm):
        cid = lax.axis_index("c")
        sid = lax.axis_index("s")
        pltpu.sync_copy(zeros_hbm, stage_v)
        pltpu.sync_copy(stage_v, acc_sm.at[pl.ds(sid * zr, zr)])

        @pl.when(cid == 0)
        def _():
            pltpu.sync_copy(dst0_hbm.at[sid], dst_v.at[pl.ds(0, k0)])

        @pl.when(cid == 1)
        def _():
            pltpu.sync_copy(dst1_hbm.at[sid], dst_v.at[pl.ds(0, k1)])

        pltpu.sync_copy(ones_hbm, ones_v)
        n_chunks = jnp.where(cid == 0, k0, k1)
        plsc.subcore_barrier()

        def body(i, carry):
            pltpu.sync_copy(ones_v, acc_sm.at[dst_v.at[i]], add=True)
            return carry

        lax.fori_loop(0, n_chunks, body, 0)
        plsc.subcore_barrier()
        pltpu.sync_copy(acc_sm.at[pl.ds(sid * zr, zr)], stage_v)

        @pl.when(cid == 0)
        def _():
            pltpu.sync_copy(stage_v, out0_hbm.at[pl.ds(sid * zr, zr)])

        @pl.when(cid == 1)
        def _():
            pltpu.sync_copy(stage_v, out1_hbm.at[pl.ds(sid * zr, zr)])

    return deg_kernel


@functools.lru_cache(maxsize=None)
def _make_agg(k0, k1, n_acc, ch):
    zr = n_acc // _SUBC
    kmax = max(k0, k1)

    @functools.partial(
        pl.kernel,
        mesh=_mesh(),
        out_type=[jax.ShapeDtypeStruct((n_acc, ch), jnp.float32),
                  jax.ShapeDtypeStruct((n_acc, ch), jnp.float32)],
        compiler_params=_SC_PARAMS,
        scratch_types=[
            pltpu.VMEM((kmax, _LANES), jnp.int32),
            pltpu.VMEM((kmax, _LANES), jnp.int32),
            pltpu.VMEM((_NBUF, _LANES, ch), jnp.float32),
            pltpu.VMEM((zr, ch), jnp.float32),
            pltpu.VMEM_SHARED((n_acc, ch), jnp.float32),
            pltpu.SemaphoreType.DMA((_NBUF,)),
        ],
    )
    def agg_kernel(g_hbm, src0_hbm, dst0_hbm, src1_hbm, dst1_hbm, zeros_hbm,
                   out0_hbm, out1_hbm, src_v, dst_v, rows_v, stage_v, acc_sm,
                   sem):
        cid = lax.axis_index("c")
        sid = lax.axis_index("s")
        pltpu.sync_copy(zeros_hbm, stage_v)
        pltpu.sync_copy(stage_v, acc_sm.at[pl.ds(sid * zr, zr)])

        @pl.when(cid == 0)
        def _():
            pltpu.sync_copy(src0_hbm.at[sid], src_v.at[pl.ds(0, k0)])
            pltpu.sync_copy(dst0_hbm.at[sid], dst_v.at[pl.ds(0, k0)])

        @pl.when(cid == 1)
        def _():
            pltpu.sync_copy(src1_hbm.at[sid], src_v.at[pl.ds(0, k1)])
            pltpu.sync_copy(dst1_hbm.at[sid], dst_v.at[pl.ds(0, k1)])

        n_chunks = jnp.where(cid == 0, k0, k1)
        plsc.subcore_barrier()

        for b in range(_NBUF):
            pltpu.async_copy(g_hbm.at[src_v.at[b]], rows_v.at[b], sem.at[b])

        def blk_body(ib, carry):
            base = ib * _NBUF
            for b in range(_NBUF):
                i = base + b
                pltpu.make_async_copy(
                    g_hbm.at[src_v.at[i]], rows_v.at[b], sem.at[b]).wait()
                pltpu.sync_copy(rows_v.at[b], acc_sm.at[dst_v.at[i]], add=True)

                @pl.when(i + _NBUF < n_chunks)
                def _():
                    pltpu.async_copy(g_hbm.at[src_v.at[i + _NBUF]],
                                     rows_v.at[b], sem.at[b])
            return carry

        lax.fori_loop(0, n_chunks // _NBUF, blk_body, 0)
        plsc.subcore_barrier()
        pltpu.sync_copy(acc_sm.at[pl.ds(sid * zr, zr)], stage_v)

        @pl.when(cid == 0)
        def _():
            pltpu.sync_copy(stage_v, out0_hbm.at[pl.ds(sid * zr, zr)])

        @pl.when(cid == 1)
        def _():
            pltpu.sync_copy(stage_v, out1_hbm.at[pl.ds(sid * zr, zr)])

    return agg_kernel


@functools.lru_cache(maxsize=None)
def _make_tc1(n, in_ch, mid, blk):
    def body(degp_ref, x_ref, w1_ref, g1_ref):
        deg = degp_ref[:, 0] + degp_ref[:, 1] + 1.0
        dinv = lax.rsqrt(deg)[:, None]
        h = jnp.dot(x_ref[...], w1_ref[...], preferred_element_type=jnp.float32)
        g1_ref[...] = h * dinv

    return pl.pallas_call(
        body,
        grid=(n // blk,),
        in_specs=[
            pl.BlockSpec((blk, 2), lambda i: (i, 0)),
            pl.BlockSpec((blk, in_ch), lambda i: (i, 0)),
            pl.BlockSpec((in_ch, mid), lambda i: (0, 0)),
        ],
        out_specs=pl.BlockSpec((blk, mid), lambda i: (i, 0)),
        out_shape=jax.ShapeDtypeStruct((n, mid), jnp.float32),
    )


@functools.lru_cache(maxsize=None)
def _make_tc2(n, mid, out_ch, blk):
    def body(degp_ref, a0_ref, a1_ref, g1_ref, b1_ref, w2_ref, g2_ref):
        deg = degp_ref[:, 0] + degp_ref[:, 1] + 1.0
        dinv = lax.rsqrt(deg)[:, None]
        agg = a0_ref[...] + a1_ref[...] + g1_ref[...]
        z = jnp.maximum(agg * dinv + b1_ref[0], 0.0)
        h2 = jnp.dot(z, w2_ref[...], preferred_element_type=jnp.float32)
        g2_ref[...] = h2 * dinv

    return pl.pallas_call(
        body,
        grid=(n // blk,),
        in_specs=[
            pl.BlockSpec((blk, 2), lambda i: (i, 0)),
            pl.BlockSpec((blk, mid), lambda i: (i, 0)),
            pl.BlockSpec((blk, mid), lambda i: (i, 0)),
            pl.BlockSpec((blk, mid), lambda i: (i, 0)),
            pl.BlockSpec((1, mid), lambda i: (0, 0)),
            pl.BlockSpec((mid, out_ch), lambda i: (0, 0)),
        ],
        out_specs=pl.BlockSpec((blk, out_ch), lambda i: (i, 0)),
        out_shape=jax.ShapeDtypeStruct((n, out_ch), jnp.float32),
    )


@functools.lru_cache(maxsize=None)
def _make_tc3(n, out_ch, blk):
    def body(degp_ref, a0_ref, a1_ref, g2_ref, b2_ref, out_ref):
        deg = degp_ref[:, 0] + degp_ref[:, 1] + 1.0
        dinv = lax.rsqrt(deg)[:, None]
        agg = a0_ref[...] + a1_ref[...] + g2_ref[...]
        o = agg * dinv + b2_ref[0]
        m = jnp.max(o, axis=1, keepdims=True)
        e = jnp.exp(o - m)
        s = jnp.sum(e, axis=1, keepdims=True)
        out_ref[...] = o - m - jnp.log(s)

    return pl.pallas_call(
        body,
        grid=(n // blk,),
        in_specs=[
            pl.BlockSpec((blk, 2), lambda i: (i, 0)),
            pl.BlockSpec((blk, out_ch), lambda i: (i, 0)),
            pl.BlockSpec((blk, out_ch), lambda i: (i, 0)),
            pl.BlockSpec((blk, out_ch), lambda i: (i, 0)),
            pl.BlockSpec((1, out_ch), lambda i: (0, 0)),
        ],
        out_specs=pl.BlockSpec((blk, out_ch), lambda i: (i, 0)),
        out_shape=jax.ShapeDtypeStruct((n, out_ch), jnp.float32),
    )


def kernel(x, edge_index, W1, b1, W2, b2):
    n, in_ch = x.shape
    mid = W1.shape[1]
    out_ch = W2.shape[1]
    n_edges = edge_index.shape[1]

    ei = edge_index.astype(jnp.int32)
    k_tot = -(-n_edges // (2 * _SUBC * _LANES)) * 2
    k0 = max(_NBUF, int(round(k_tot * _CORE0_FRAC / _NBUF)) * _NBUF)
    k1 = max(_NBUF, -(-(k_tot - k0) // _NBUF) * _NBUF)
    e0 = _SUBC * k0 * _LANES
    e_pad = e0 + _SUBC * k1 * _LANES
    pad = e_pad - n_edges
    src = jnp.concatenate([ei[0], jnp.zeros((pad,), jnp.int32)])
    dst = jnp.concatenate([ei[1], jnp.full((pad,), n, jnp.int32)])
    src0 = src[:e0].reshape(_SUBC, k0, _LANES)
    dst0 = dst[:e0].reshape(_SUBC, k0, _LANES)
    src1 = src[e0:].reshape(_SUBC, k1, _LANES)
    dst1 = dst[e0:].reshape(_SUBC, k1, _LANES)

    n_acc = ((n + 1 + 127) // 128) * 128

    zr = n_acc // _SUBC
    ones = jnp.ones((_LANES,), jnp.float32)
    d0, d1 = _make_deg(k0, k1, n_acc)(dst0, dst1, ones,
                                      jnp.zeros((zr,), jnp.float32))
    degp2 = jnp.stack([d0[:n], d1[:n]], axis=1)

    g1 = _make_tc1(n, in_ch, mid, _ROW_BLK)(degp2, x, W1)
    a0, a1 = _make_agg(k0, k1, n_acc, mid)(
        g1, src0, dst0, src1, dst1, jnp.zeros((zr, mid), jnp.float32))
    g2 = _make_tc2(n, mid, out_ch, _ROW_BLK)(
        degp2, a0, a1, g1, b1.reshape(1, -1), W2)
    a0, a1 = _make_agg(k0, k1, n_acc, out_ch)(
        g2, src0, dst0, src1, dst1, jnp.zeros((zr, out_ch), jnp.float32))
    return _make_tc3(n, out_ch, _ROW_BLK)(
        degp2, a0, a1, g2, b2.reshape(1, -1))


# speedup vs baseline: 1.2252x; 1.2252x over previous
import functools

import jax
import jax.numpy as jnp
from jax import lax
from jax.experimental import pallas as pl
from jax.experimental.pallas import tpu as pltpu
from jax.experimental.pallas import tpu_sc as plsc

_LANES = 128
_SUBC = 16
_ROW_BLK = 2000
_NBUF = 4
_CORE0_FRAC = 0.75


def _mesh():
    return plsc.VectorSubcoreMesh(core_axis_name="c", subcore_axis_name="s")


_SC_PARAMS = pltpu.CompilerParams(use_tc_tiling_on_sc=False)


@functools.lru_cache(maxsize=None)
def _make_deg(k0, k1, n_acc):
    zr = n_acc // _SUBC
    kmax = max(k0, k1)

    @functools.partial(
        pl.kernel,
        mesh=_mesh(),
        out_type=[jax.ShapeDtypeStruct((n_acc,), jnp.float32),
                  jax.ShapeDtypeStruct((n_acc,), jnp.float32)],
        compiler_params=_SC_PARAMS,
        scratch_types=[
            pltpu.VMEM((kmax, _LANES), jnp.int32),
            pltpu.VMEM((_LANES,), jnp.float32),
            pltpu.VMEM((zr,), jnp.float32),
            pltpu.VMEM_SHARED((n_acc,), jnp.float32),
        ],
    )
    def deg_kernel(dst0_hbm, dst1_hbm, ones_hbm, zeros_hbm, out0_hbm, out1_hbm,
                   dst_v, ones_v, stage_v, acc_sm):
        cid = lax.axis_index("c")
        sid = lax.axis_index("s")
        pltpu.sync_copy(zeros_hbm, stage_v)
        pltpu.sync_copy(stage_v, acc_sm.at[pl.ds(sid * zr, zr)])

        @pl.when(cid == 0)
        def _():
            pltpu.sync_copy(dst0_hbm.at[sid], dst_v.at[pl.ds(0, k0)])

        @pl.when(cid == 1)
        def _():
            pltpu.sync_copy(dst1_hbm.at[sid], dst_v.at[pl.ds(0, k1)])

        pltpu.sync_copy(ones_hbm, ones_v)
        n_chunks = jnp.where(cid == 0, k0, k1)
        plsc.subcore_barrier()

        def body(i, carry):
            pltpu.sync_copy(ones_v, acc_sm.at[dst_v.at[i]], add=True)
            return carry

        lax.fori_loop(0, n_chunks, body, 0)
        plsc.subcore_barrier()
        pltpu.sync_copy(acc_sm.at[pl.ds(sid * zr, zr)], stage_v)

        @pl.when(cid == 0)
        def _():
            pltpu.sync_copy(stage_v, out0_hbm.at[pl.ds(sid * zr, zr)])

        @pl.when(cid == 1)
        def _():
            pltpu.sync_copy(stage_v, out1_hbm.at[pl.ds(sid * zr, zr)])

    return deg_kernel


@functools.lru_cache(maxsize=None)
def _make_agg(k0, k1, n_acc, ch):
    zr = n_acc // _SUBC
    kmax = max(k0, k1)

    @functools.partial(
        pl.kernel,
        mesh=_mesh(),
        out_type=[jax.ShapeDtypeStruct((n_acc, ch), jnp.float32),
                  jax.ShapeDtypeStruct((n_acc, ch), jnp.float32)],
        compiler_params=_SC_PARAMS,
        scratch_types=[
            pltpu.VMEM((kmax, _LANES), jnp.int32),
            pltpu.VMEM((kmax, _LANES), jnp.int32),
            pltpu.VMEM((_NBUF, _LANES, ch), jnp.float32),
            pltpu.VMEM((zr, ch), jnp.float32),
            pltpu.VMEM_SHARED((n_acc, ch), jnp.float32),
            pltpu.SemaphoreType.DMA((_NBUF,)),
        ],
    )
    def agg_kernel(g_hbm, src0_hbm, dst0_hbm, src1_hbm, dst1_hbm, zeros_hbm,
                   out0_hbm, out1_hbm, src_v, dst_v, rows_v, stage_v, acc_sm,
                   sem):
        cid = lax.axis_index("c")
        sid = lax.axis_index("s")
        pltpu.sync_copy(zeros_hbm, stage_v)
        pltpu.sync_copy(stage_v, acc_sm.at[pl.ds(sid * zr, zr)])

        @pl.when(cid == 0)
        def _():
            pltpu.sync_copy(src0_hbm.at[sid], src_v.at[pl.ds(0, k0)])
            pltpu.sync_copy(dst0_hbm.at[sid], dst_v.at[pl.ds(0, k0)])

        @pl.when(cid == 1)
        def _():
            pltpu.sync_copy(src1_hbm.at[sid], src_v.at[pl.ds(0, k1)])
            pltpu.sync_copy(dst1_hbm.at[sid], dst_v.at[pl.ds(0, k1)])

        n_chunks = jnp.where(cid == 0, k0, k1)
        plsc.subcore_barrier()

        for b in range(_NBUF):
            pltpu.async_copy(g_hbm.at[src_v.at[b]], rows_v.at[b], sem.at[b])

        def blk_body(ib, carry):
            base = ib * _NBUF
            for b in range(_NBUF):
                i = base + b
                pltpu.make_async_copy(
                    g_hbm.at[src_v.at[i]], rows_v.at[b], sem.at[b]).wait()
                pltpu.sync_copy(rows_v.at[b], acc_sm.at[dst_v.at[i]], add=True)

                @pl.when(i + _NBUF < n_chunks)
                def _():
                    pltpu.async_copy(g_hbm.at[src_v.at[i + _NBUF]],
                                     rows_v.at[b], sem.at[b])
            return carry

        lax.fori_loop(0, n_chunks // _NBUF, blk_body, 0)
        plsc.subcore_barrier()
        pltpu.sync_copy(acc_sm.at[pl.ds(sid * zr, zr)], stage_v)

        @pl.when(cid == 0)
        def _():
            pltpu.sync_copy(stage_v, out0_hbm.at[pl.ds(sid * zr, zr)])

        @pl.when(cid == 1)
        def _():
            pltpu.sync_copy(stage_v, out1_hbm.at[pl.ds(sid * zr, zr)])

    return agg_kernel


@functools.lru_cache(maxsize=None)
def _make_tc1(n, in_ch, mid, blk):
    def body(degp_ref, x_ref, w1_ref, g1_ref):
        deg = degp_ref[:, 0] + degp_ref[:, 1] + 1.0
        dinv = lax.rsqrt(deg)[:, None]
        h = jnp.dot(x_ref[...], w1_ref[...], preferred_element_type=jnp.float32)
        g1_ref[...] = h * dinv

    return pl.pallas_call(
        body,
        grid=(n // blk,),
        in_specs=[
            pl.BlockSpec((blk, 2), lambda i: (i, 0)),
            pl.BlockSpec((blk, in_ch), lambda i: (i, 0)),
            pl.BlockSpec((in_ch, mid), lambda i: (0, 0)),
        ],
        out_specs=pl.BlockSpec((blk, mid), lambda i: (i, 0)),
        out_shape=jax.ShapeDtypeStruct((n, mid), jnp.float32),
    )


@functools.lru_cache(maxsize=None)
def _make_tc2(n, mid, out_ch, blk):
    def body(degp_ref, a0_ref, a1_ref, g1_ref, b1_ref, w2_ref, g2_ref):
        deg = degp_ref[:, 0] + degp_ref[:, 1] + 1.0
        dinv = lax.rsqrt(deg)[:, None]
        agg = a0_ref[...] + a1_ref[...] + g1_ref[...]
        z = jnp.maximum(agg * dinv + b1_ref[0], 0.0)
        h2 = jnp.dot(z, w2_ref[...], preferred_element_type=jnp.float32)
        g2_ref[...] = h2 * dinv

    return pl.pallas_call(
        body,
        grid=(n // blk,),
        in_specs=[
            pl.BlockSpec((blk, 2), lambda i: (i, 0)),
            pl.BlockSpec((blk, mid), lambda i: (i, 0)),
            pl.BlockSpec((blk, mid), lambda i: (i, 0)),
            pl.BlockSpec((blk, mid), lambda i: (i, 0)),
            pl.BlockSpec((1, mid), lambda i: (0, 0)),
            pl.BlockSpec((mid, out_ch), lambda i: (0, 0)),
        ],
        out_specs=pl.BlockSpec((blk, out_ch), lambda i: (i, 0)),
        out_shape=jax.ShapeDtypeStruct((n, out_ch), jnp.float32),
    )


@functools.lru_cache(maxsize=None)
def _make_tc3(n, out_ch, blk):
    def body(degp_ref, a0_ref, a1_ref, g2_ref, b2_ref, out_ref):
        deg = degp_ref[:, 0] + degp_ref[:, 1] + 1.0
        dinv = lax.rsqrt(deg)[:, None]
        agg = a0_ref[...] + a1_ref[...] + g2_ref[...]
        o = agg * dinv + b2_ref[0]
        m = jnp.max(o, axis=1, keepdims=True)
        e = jnp.exp(o - m)
        s = jnp.sum(e, axis=1, keepdims=True)
        out_ref[...] = o - m - jnp.log(s)

    return pl.pallas_call(
        body,
        grid=(n // blk,),
        in_specs=[
            pl.BlockSpec((blk, 2), lambda i: (i, 0)),
            pl.BlockSpec((blk, out_ch), lambda i: (i, 0)),
            pl.BlockSpec((blk, out_ch), lambda i: (i, 0)),
            pl.BlockSpec((blk, out_ch), lambda i: (i, 0)),
            pl.BlockSpec((1, out_ch), lambda i: (0, 0)),
        ],
        out_specs=pl.BlockSpec((blk, out_ch), lambda i: (i, 0)),
        out_shape=jax.ShapeDtypeStruct((n, out_ch), jnp.float32),
    )


def kernel(x, edge_index, W1, b1, W2, b2):
    n, in_ch = x.shape
    mid = W1.shape[1]
    out_ch = W2.shape[1]
    n_edges = edge_index.shape[1]

    ei = edge_index.astype(jnp.int32)
    k_tot = -(-n_edges // (2 * _SUBC * _LANES)) * 2
    k0 = max(_NBUF, int(round(k_tot * _CORE0_FRAC / _NBUF)) * _NBUF)
    k1 = max(_NBUF, -(-(k_tot - k0) // _NBUF) * _NBUF)
    e0 = _SUBC * k0 * _LANES
    e_pad = e0 + _SUBC * k1 * _LANES
    pad = e_pad - n_edges
    src = jnp.concatenate([ei[0], jnp.zeros((pad,), jnp.int32)])
    dst = jnp.concatenate([ei[1], jnp.full((pad,), n, jnp.int32)])
    src0 = src[:e0].reshape(_SUBC, k0, _LANES)
    dst0 = dst[:e0].reshape(_SUBC, k0, _LANES)
    src1 = src[e0:].reshape(_SUBC, k1, _LANES)
    dst1 = dst[e0:].reshape(_SUBC, k1, _LANES)

    n_acc = ((n + 1 + 127) // 128) * 128

    zr = n_acc // _SUBC
    ones = jnp.ones((_LANES,), jnp.float32)
    d0, d1 = _make_deg(k0, k1, n_acc)(dst0, dst1, ones,
                                      jnp.zeros((zr,), jnp.float32))
    degp2 = jnp.stack([d0[:n], d1[:n]], axis=1)

    g1 = _make_tc1(n, in_ch, mid, _ROW_BLK)(degp2, x, W1)
    a0, a1 = _make_agg(k0, k1, n_acc, mid)(
        g1, src0, dst0, src1, dst1, jnp.zeros((zr, mid), jnp.float32))
    g2 = _make_tc2(n, mid, out_ch, _ROW_BLK)(
        degp2, a0, a1, g1, b1.reshape(1, -1), W2)
    a0, a1 = _make_agg(k0, k1, n_acc, out_ch)(
        g2, src0, dst0, src1, dst1, jnp.zeros((zr, out_ch), jnp.float32))
    return _make_tc3(n, out_ch, _ROW_BLK)(
        degp2, a0, a1, g2, b2.reshape(1, -1))
